# baseline (speedup 1.0000x reference)
import functools

import jax
import jax.numpy as jnp
from jax import lax
from jax.experimental import pallas as pl
from jax.experimental.pallas import tpu as pltpu
from jax.experimental.pallas import tpu_sc as plsc

EMBED_DIM = 128
ATTN_DIM = 768

_NC = 2
_NS = 16
_NW = _NC * _NS

_CHUNK = 128
_NBUF = 4


def _make_sc_gather(n_tokens: int):
    per_w = n_tokens // _NW
    chunks = per_w // _CHUNK
    assert chunks % _NBUF == 0

    mesh = plsc.VectorSubcoreMesh(core_axis_name="c", subcore_axis_name="s")

    @functools.partial(
        pl.kernel,
        mesh=mesh,
        out_type=jax.ShapeDtypeStruct((n_tokens, EMBED_DIM), jnp.float32),
        scratch_types=[
            pltpu.VMEM((chunks, _CHUNK), jnp.int32),
            *[pltpu.VMEM((_CHUNK, EMBED_DIM), jnp.float32)] * _NBUF,
            *[pltpu.SemaphoreType.DMA] * (2 * _NBUF),
        ],
    )
    def sc_gather(table_hbm, idx_hbm, out_hbm, idx_v, *bufs_sems):
        rows = bufs_sems[:_NBUF]
        gsem = bufs_sems[_NBUF:2 * _NBUF]
        wsem = bufs_sems[2 * _NBUF:]
        wid = lax.axis_index("s") * _NC + lax.axis_index("c")
        row_base = wid * chunks
        pltpu.sync_copy(idx_hbm.at[wid], idx_v)

        def gather(b, g):
            pltpu.make_async_copy(
                table_hbm.at[idx_v.at[g]], rows[b], gsem[b]).start()

        def put(b, g):
            tok = (row_base + g) * _CHUNK
            pltpu.make_async_copy(
                rows[b], out_hbm.at[pl.ds(tok, _CHUNK)], wsem[b]).start()

        for b in range(_NBUF):
            gather(b, b)

        def body(gg, carry):
            for b in range(_NBUF):
                g = gg * _NBUF + b
                pltpu.make_async_copy(
                    table_hbm.at[idx_v.at[g]], rows[b], gsem[b]).wait()
                put(b, g)
            for b in range(_NBUF):
                g2 = (gg + 1) * _NBUF + b
                tok = (row_base + g2 - _NBUF) * _CHUNK
                pltpu.make_async_copy(
                    rows[b], out_hbm.at[pl.ds(tok, _CHUNK)], wsem[b]).wait()

                @pl.when(g2 < chunks)
                def _():
                    gather(b, g2)
            return carry

        lax.fori_loop(0, chunks // _NBUF, body, 0)

    return sc_gather


def _mm_body(emb_ref, w_ref, out_ref):
    out_ref[...] = lax.dot_general(
        emb_ref[...], w_ref[...],
        dimension_numbers=(((1,), (1,)), ((), ())),
        preferred_element_type=jnp.float32,
    )


def _project(emb, w, tile: int):
    n = emb.shape[0]
    return pl.pallas_call(
        _mm_body,
        grid=(n // tile,),
        in_specs=[
            pl.BlockSpec((tile, EMBED_DIM), lambda i: (i, 0)),
            pl.BlockSpec((ATTN_DIM, EMBED_DIM), lambda i: (0, 0)),
        ],
        out_specs=pl.BlockSpec((tile, ATTN_DIM), lambda i: (i, 0)),
        out_shape=jax.ShapeDtypeStruct((n, ATTN_DIM), jnp.float32),
        compiler_params=pltpu.CompilerParams(
            dimension_semantics=("parallel",),
        ),
    )(emb, w)


def _mm_body_alias(emb_ref, w_ref, prev_ref, out_ref):
    del prev_ref
    _mm_body(emb_ref, w_ref, out_ref)


def _project_half(emb, w, n: int, tile: int, offset_tiles: int, prev=None):
    half = emb.shape[0]
    in_specs = [
        pl.BlockSpec((tile, EMBED_DIM), lambda i: (i, 0)),
        pl.BlockSpec((ATTN_DIM, EMBED_DIM), lambda i: (0, 0)),
    ]
    args = [emb, w]
    body = _mm_body
    aliases = {}
    if prev is not None:
        in_specs.append(pl.BlockSpec(memory_space=pl.ANY))
        args.append(prev)
        body = _mm_body_alias
        aliases = {2: 0}
    return pl.pallas_call(
        body,
        grid=(half // tile,),
        in_specs=in_specs,
        out_specs=pl.BlockSpec(
            (tile, ATTN_DIM), lambda i: (i + offset_tiles, 0)),
        out_shape=jax.ShapeDtypeStruct((n, ATTN_DIM), jnp.float32),
        input_output_aliases=aliases,
        compiler_params=pltpu.CompilerParams(
            dimension_semantics=("arbitrary",),
        ),
    )(*args)


def kernel(x, table, W):
    b, s = x.shape
    n = b * s
    tile = 8192
    half = n // 2
    chunks = half // _NW // _CHUNK
    idx3 = x.reshape(2, _NW, chunks, _CHUNK).astype(jnp.int32)
    gather = _make_sc_gather(half)
    emb0 = gather(table, idx3[0])
    emb1 = gather(table, idx3[1])
    out0 = _project_half(emb0, W, n, tile, 0)
    out = _project_half(emb1, W, n, tile, n // 2 // tile, prev=out0)
    return out.reshape(b, s, ATTN_DIM)


# speedup vs baseline: 1.0044x; 1.0044x over previous
import functools

import jax
import jax.numpy as jnp
from jax import lax
from jax.experimental import pallas as pl
from jax.experimental.pallas import tpu as pltpu
from jax.experimental.pallas import tpu_sc as plsc

EMBED_DIM = 128
ATTN_DIM = 768

_NC = 2
_NS = 16
_NW = _NC * _NS

_CHUNK = 128
_NBUF = 4


def _make_sc_gather(n_tokens: int):
    per_w = n_tokens // _NW
    chunks = per_w // _CHUNK
    assert chunks % _NBUF == 0

    mesh = plsc.VectorSubcoreMesh(core_axis_name="c", subcore_axis_name="s")

    @functools.partial(
        pl.kernel,
        mesh=mesh,
        out_type=jax.ShapeDtypeStruct((n_tokens, EMBED_DIM), jnp.float32),
        scratch_types=[
            pltpu.VMEM((chunks, _CHUNK), jnp.int32),
            *[pltpu.VMEM((_CHUNK, EMBED_DIM), jnp.float32)] * _NBUF,
            *[pltpu.SemaphoreType.DMA] * (2 * _NBUF),
        ],
    )
    def sc_gather(table_hbm, idx_hbm, out_hbm, idx_v, *bufs_sems):
        rows = bufs_sems[:_NBUF]
        gsem = bufs_sems[_NBUF:2 * _NBUF]
        wsem = bufs_sems[2 * _NBUF:]
        wid = lax.axis_index("s") * _NC + lax.axis_index("c")
        row_base = wid * chunks
        pltpu.sync_copy(idx_hbm.at[wid], idx_v)

        def gather(b, g):
            pltpu.make_async_copy(
                table_hbm.at[idx_v.at[g]], rows[b], gsem[b]).start()

        def put(b, g):
            tok = (row_base + g) * _CHUNK
            pltpu.make_async_copy(
                rows[b], out_hbm.at[pl.ds(tok, _CHUNK)], wsem[b]).start()

        for b in range(_NBUF):
            gather(b, b)

        def body(gg, carry):
            for b in range(_NBUF):
                g = gg * _NBUF + b
                pltpu.make_async_copy(
                    table_hbm.at[idx_v.at[g]], rows[b], gsem[b]).wait()
                put(b, g)
            for b in range(_NBUF):
                g2 = (gg + 1) * _NBUF + b
                tok = (row_base + g2 - _NBUF) * _CHUNK
                pltpu.make_async_copy(
                    rows[b], out_hbm.at[pl.ds(tok, _CHUNK)], wsem[b]).wait()

                @pl.when(g2 < chunks)
                def _():
                    gather(b, g2)
            return carry

        lax.fori_loop(0, chunks // _NBUF, body, 0)

    return sc_gather


def _mm_body(emb_ref, w_ref, out_ref):
    out_ref[...] = lax.dot_general(
        emb_ref[...], w_ref[...],
        dimension_numbers=(((1,), (1,)), ((), ())),
        preferred_element_type=jnp.float32,
    )


def _project(emb, w, tile: int):
    n = emb.shape[0]
    return pl.pallas_call(
        _mm_body,
        grid=(n // tile,),
        in_specs=[
            pl.BlockSpec((tile, EMBED_DIM), lambda i: (i, 0)),
            pl.BlockSpec((ATTN_DIM, EMBED_DIM), lambda i: (0, 0)),
        ],
        out_specs=pl.BlockSpec((tile, ATTN_DIM), lambda i: (i, 0)),
        out_shape=jax.ShapeDtypeStruct((n, ATTN_DIM), jnp.float32),
        compiler_params=pltpu.CompilerParams(
            dimension_semantics=("parallel",),
        ),
    )(emb, w)


def kernel(x, table, W):
    b, s = x.shape
    n = b * s
    chunks = n // _NW // _CHUNK
    idx3 = x.reshape(_NW, chunks, _CHUNK).astype(jnp.int32)
    emb = _make_sc_gather(n)(table, idx3)
    out = _project(emb, W, tile=8192)
    return out.reshape(b, s, ATTN_DIM)
